# baseline (speedup 1.0000x reference)
import jax
import jax.numpy as jnp
from jax.experimental import pallas as pl


def _mask_cast_kernel(mask_ref, out_ref):
    out_ref[...] = mask_ref[...] != 0.0


def kernel(x, v, mask):
    b, one, l = mask.shape
    m2 = mask.reshape(b * one, l)
    mask_bool = pl.pallas_call(
        _mask_cast_kernel,
        out_shape=jax.ShapeDtypeStruct(m2.shape, jnp.bool_),
    )(m2)
    return (x, v, mask_bool.reshape(b, one, l))
